# Initial kernel scaffold; baseline (speedup 1.0000x reference)
import jax
import jax.numpy as jnp
from jax.experimental import pallas as pl


def kernel(x):
    raise NotImplementedError("write your pallas kernel here")


# speedup vs baseline: 1.4086x; 1.4086x over previous
import jax
import jax.numpy as jnp
from jax.experimental import pallas as pl
from jax.experimental.pallas import tpu as pltpu

_B = 64
_N = 576
_NUM_MASK = 432
_TAIL = _N - _NUM_MASK

_ROT = ((13, 15, 26, 6), (17, 29, 16, 24))


def _threefry_keys():
    u32 = jnp.uint32
    row = jax.lax.broadcasted_iota(jnp.int32, (_B, _N), 0)
    col = jax.lax.broadcasted_iota(jnp.int32, (_B, _N), 1)
    x1 = (row * _N + col).astype(u32)
    x0 = jnp.zeros((_B, _N), u32)

    ks0 = u32(0)
    ks1 = u32(42)
    ks2 = u32(0 ^ 42 ^ 0x1BD11BDA)
    inj = ((ks1, ks2), (ks2, ks0), (ks0, ks1), (ks1, ks2), (ks2, ks0))

    x0 = x0 + ks0
    x1 = x1 + ks1
    for i in range(5):
        for r in _ROT[i % 2]:
            x0 = x0 + x1
            x1 = (x1 << u32(r)) | (x1 >> u32(32 - r))
            x1 = x0 ^ x1
        a, b = inj[i]
        x0 = x0 + a
        x1 = x1 + b + u32(i + 1)

    bits = x0 ^ x1
    return (bits >> u32(9)).astype(jnp.int32)


def _mask_kernel(out_ref, keys_ref):
    keys_ref[:, :] = _threefry_keys()
    keys = keys_ref[:, :]
    lane = jax.lax.broadcasted_iota(jnp.int32, (_B, _N), 1)
    acc = jnp.zeros((_B, _N), jnp.int32)
    for j in range(_NUM_MASK, _N):
        t = keys_ref[:, j : j + 1]
        rank = jnp.sum((keys < t).astype(jnp.int32), axis=1, keepdims=True)
        acc = acc + (lane == rank).astype(jnp.int32)
    out_ref[:, :] = acc == 0


def _build_mask():
    return pl.pallas_call(
        _mask_kernel,
        out_shape=jax.ShapeDtypeStruct((_B, _N), jnp.bool_),
        scratch_shapes=[pltpu.VMEM((_B, _N), jnp.int32)],
    )()


def kernel(x):
    assert x.shape[0] == _B
    return _build_mask()


# speedup vs baseline: 3.3583x; 2.3842x over previous
import jax
import jax.numpy as jnp
from jax.experimental import pallas as pl
from jax.experimental.pallas import tpu as pltpu

_B = 64
_N = 576
_NUM_MASK = 432
_TAIL = _N - _NUM_MASK

_ROT = ((13, 15, 26, 6), (17, 29, 16, 24))


def _threefry_keys():
    u32 = jnp.uint32
    row = jax.lax.broadcasted_iota(jnp.int32, (_B, _N), 0)
    col = jax.lax.broadcasted_iota(jnp.int32, (_B, _N), 1)
    x1 = (row * _N + col).astype(u32)
    x0 = jnp.zeros((_B, _N), u32)

    ks0 = u32(0)
    ks1 = u32(42)
    ks2 = u32(0 ^ 42 ^ 0x1BD11BDA)
    inj = ((ks1, ks2), (ks2, ks0), (ks0, ks1), (ks1, ks2), (ks2, ks0))

    x0 = x0 + ks0
    x1 = x1 + ks1
    for i in range(5):
        for r in _ROT[i % 2]:
            x0 = x0 + x1
            x1 = (x1 << u32(r)) | (x1 >> u32(32 - r))
            x1 = x0 ^ x1
        a, b = inj[i]
        x0 = x0 + a
        x1 = x1 + b + u32(i + 1)

    bits = x0 ^ x1
    return (bits >> u32(9)).astype(jnp.int32)


def _mask_kernel(out_ref, keys_ref):
    keys_ref[:, :] = _threefry_keys()
    keys = keys_ref[:, :]
    lane = jax.lax.broadcasted_iota(jnp.int32, (_B, _N), 1).astype(jnp.int16)
    ones_col = jnp.ones((_N, 1), jnp.float32)
    acc = jnp.zeros((_B, _N), jnp.int16)
    for j in range(_NUM_MASK, _N):
        t = keys_ref[:, j : j + 1]
        cmp = (keys < t).astype(jnp.float32)
        rank = jax.lax.dot_general(
            cmp, ones_col, (((1,), (0,)), ((), ())),
            preferred_element_type=jnp.float32,
        ).astype(jnp.int16)
        acc = acc + (lane == rank).astype(jnp.int16)
    out_ref[:, :] = acc == 0


def _build_mask():
    return pl.pallas_call(
        _mask_kernel,
        out_shape=jax.ShapeDtypeStruct((_B, _N), jnp.bool_),
        scratch_shapes=[pltpu.VMEM((_B, _N), jnp.int32)],
    )()


def kernel(x):
    assert x.shape[0] == _B
    return _build_mask()
